# baseline (speedup 1.0000x reference)
import functools

import jax
import jax.numpy as jnp
from jax import lax
from jax.experimental import pallas as pl
from jax.experimental.pallas import tpu as pltpu
from jax.experimental.pallas import tpu_sc as plsc

B = 16384
BAG = 50
TD = 64
CD = 32

NC = 2
NS = 16
NW = NC * NS
EPW = B // NW

CHUNK = 8
RPC = CHUNK * BAG
NCHUNK = EPW // CHUNK
CGRP = 64

_GDN = lax.GatherDimensionNumbers(offset_dims=(), collapsed_slice_dims=(0,),
                                  start_index_map=(0,))


def _lane_pick(vec, lane):
    idx = jnp.full((16, 1), lane, jnp.int32)
    return lax.gather(vec, idx, dimension_numbers=_GDN, slice_sizes=(1,),
                      mode=lax.GatherScatterMode.PROMISE_IN_BOUNDS)


def _sc_body(pidx_hbm, pcol_hbm, qidx_hbm, ccol_hbm, ttab_hbm, ctab_hbm,
             sum_hbm, cemb_hbm,
             idx_v, pcol_v, qidx_v, ccol_v, rows_v, crows_v, out_v, cout_v,
             sem, csem):
    c = lax.axis_index("c")
    s = lax.axis_index("s")
    wid = c * NS + s
    base = pl.multiple_of(wid * EPW, EPW)

    pltpu.sync_copy(qidx_hbm.at[pl.ds(pl.multiple_of(wid * 8, 8), 8)], qidx_v)
    pltpu.sync_copy(ccol_hbm.at[pl.ds(base, EPW)], ccol_v)
    iota = lax.broadcasted_iota(jnp.int32, (16,), 0)

    def chunk_body(k, carry):
        ebase = pl.multiple_of(base + k * CHUNK, CHUNK)
        pltpu.sync_copy(pidx_hbm.at[pl.ds(ebase, CHUNK)], idx_v)
        pltpu.sync_copy(pcol_hbm.at[pl.ds(pl.multiple_of(ebase * BAG, 8), RPC)],
                        pcol_v)
        descs = [pltpu.async_copy(ttab_hbm.at[idx_v.at[e]],
                                  rows_v.at[pl.ds(e * BAG, BAG)], sem)
                 for e in range(CHUNK)]

        @pl.when((k & 7) == 0)
        def _():
            pltpu.async_copy(ctab_hbm.at[qidx_v.at[k >> 3]], crows_v,
                             csem).wait()

        for d_ in descs:
            d_.wait()

        def elem_body(e, c2):
            accs = [jnp.zeros((16,), jnp.float32) for _ in range(TD // 16)]
            for l in range(BAG):
                j = e * BAG + l
                g16 = pl.multiple_of((j >> 4) << 4, 16)
                pv = pcol_v[pl.ds(g16, 16)]
                pb = _lane_pick(pv, j & 15)
                rowi = jnp.full((16,), j, jnp.int32)
                for d in range(TD // 16):
                    accs[d] = accs[d] + plsc.load_gather(
                        rows_v, [rowi, pb + (iota + d * 16)])
            for d in range(TD // 16):
                out_v[e, pl.ds(d * 16, 16)] = accs[d]
            eg = k * CHUNK + e
            cg = pl.multiple_of((eg >> 4) << 4, 16)
            cv = ccol_v[pl.ds(cg, 16)]
            cb = _lane_pick(cv, eg & 15)
            crow = jnp.full((16,), ((k & 7) << 3) + e, jnp.int32)
            for q in range(CD // 16):
                cout_v[e, pl.ds(q * 16, 16)] = plsc.load_gather(
                    crows_v, [crow, cb + (iota + q * 16)])
            return c2

        lax.fori_loop(0, CHUNK, elem_body, 0)
        pltpu.sync_copy(out_v, sum_hbm.at[pl.ds(ebase, CHUNK)])
        pltpu.sync_copy(cout_v, cemb_hbm.at[pl.ds(ebase, CHUNK)])
        return carry

    lax.fori_loop(0, NCHUNK, chunk_body, 0)


_sc_gather = functools.partial(
    pl.kernel,
    out_type=[
        jax.ShapeDtypeStruct((B, TD), jnp.float32),
        jax.ShapeDtypeStruct((B, CD), jnp.float32),
    ],
    mesh=plsc.VectorSubcoreMesh(core_axis_name="c", subcore_axis_name="s"),
    compiler_params=pltpu.CompilerParams(use_tc_tiling_on_sc=True,
                                         needs_layout_passes=False),
    scratch_types=[
        pltpu.VMEM((CHUNK, BAG), jnp.int32),
        pltpu.VMEM((RPC,), jnp.int32),
        pltpu.VMEM((8, CGRP), jnp.int32),
        pltpu.VMEM((EPW,), jnp.int32),
        pltpu.VMEM((RPC, 128), jnp.float32),
        pltpu.VMEM((CGRP, 128), jnp.float32),
        pltpu.VMEM((CHUNK, TD), jnp.float32),
        pltpu.VMEM((CHUNK, CD), jnp.float32),
        pltpu.SemaphoreType.DMA,
        pltpu.SemaphoreType.DMA,
    ],
)(_sc_body)


MLP_BLK = 2048


def _mlp_body(s_ref, c_ref, t_ref, a1_ref, a2_ref, w2_ref, w3_ref,
              b1_ref, b2_ref, b3_ref, o_ref):
    cnt = jnp.sum((t_ref[...] != 0).astype(jnp.float32), axis=1,
                  keepdims=True)
    t = s_ref[...] / jnp.maximum(cnt, 1.0)
    hp = jax.lax.Precision.HIGHEST
    h = jnp.dot(t, a1_ref[...], precision=hp)
    h = h + jnp.dot(c_ref[...], a2_ref[...], precision=hp)
    h = jnp.maximum(h + b1_ref[...], 0.0)
    h = jnp.maximum(jnp.dot(h, w2_ref[...], precision=hp) + b2_ref[...], 0.0)
    o_ref[...] = jnp.dot(h, w3_ref[...], precision=hp) + b3_ref[...]


def _mlp(sums, cemb, text, a1, a2, w2t, w3t, b1, b2, b3):
    grid = B // MLP_BLK
    h1 = b1.shape[-1]
    h2 = b2.shape[-1]
    return pl.pallas_call(
        _mlp_body,
        grid=(grid,),
        in_specs=[
            pl.BlockSpec((MLP_BLK, TD), lambda i: (i, 0)),
            pl.BlockSpec((MLP_BLK, CD), lambda i: (i, 0)),
            pl.BlockSpec((MLP_BLK, BAG), lambda i: (i, 0)),
            pl.BlockSpec((TD, h1), lambda i: (0, 0)),
            pl.BlockSpec((CD, h1), lambda i: (0, 0)),
            pl.BlockSpec((h1, h2), lambda i: (0, 0)),
            pl.BlockSpec((h2, TD), lambda i: (0, 0)),
            pl.BlockSpec((1, h1), lambda i: (0, 0)),
            pl.BlockSpec((1, h2), lambda i: (0, 0)),
            pl.BlockSpec((1, TD), lambda i: (0, 0)),
        ],
        out_specs=pl.BlockSpec((MLP_BLK, TD), lambda i: (i, 0)),
        out_shape=jax.ShapeDtypeStruct((B, TD), jnp.float32),
    )(sums, cemb, text, a1, a2, w2t, w3t, b1, b2, b3)


def kernel(text, category, text_table, cat_table, W1, b1, W2, b2, W3, b3):
    text = text.astype(jnp.int32)
    category = category.astype(jnp.int32)
    pidx = text >> 1
    pcol = ((text & 1) << 6).reshape(-1)
    qidx = (category >> 2).reshape(B // CGRP, CGRP)
    ccol = (category & 3) << 5
    ttab2 = text_table.reshape(text_table.shape[0] // 2, 2 * TD)
    ctab2 = cat_table.reshape(cat_table.shape[0] // 4, 4 * CD)
    sums, cemb = _sc_gather(pidx, pcol, qidx, ccol, ttab2, ctab2)
    a1 = W1.T[:TD, :]
    a2 = W1.T[TD:, :]
    return _mlp(sums, cemb, text, a1, a2, W2.T, W3.T,
                b1.reshape(1, -1), b2.reshape(1, -1), b3.reshape(1, -1))


# speedup vs baseline: 1.3082x; 1.3082x over previous
import functools

import jax
import jax.numpy as jnp
from jax import lax
from jax.experimental import pallas as pl
from jax.experimental.pallas import tpu as pltpu
from jax.experimental.pallas import tpu_sc as plsc

B = 16384
BAG = 50
TD = 64
CD = 32

NC = 2
NS = 16
NW = NC * NS
EPW = B // NW

CHUNK = 8
RPC = CHUNK * BAG
NCHUNK = EPW // CHUNK
CAT_GB = 64


def _sc_body(text_hbm, cat_hbm, ttab_hbm, ctab_hbm, sum_hbm, cemb_hbm,
             idx0, idx1, rows0, rows1, out0, out1, cidx_v, crows_v,
             sem0, sem1, osem):
    c = lax.axis_index("c")
    s = lax.axis_index("s")
    wid = c * NS + s
    base = pl.multiple_of(wid * EPW, EPW)

    crow0 = pl.multiple_of(wid * (EPW // CAT_GB), EPW // CAT_GB)
    pltpu.sync_copy(cat_hbm.at[pl.ds(crow0, EPW // CAT_GB)], cidx_v)
    cds = [pltpu.async_copy(ctab_hbm.at[cidx_v.at[j]],
                            crows_v.at[pl.ds(j * CAT_GB, CAT_GB)], sem0)
           for j in range(EPW // CAT_GB)]
    for d in cds:
        d.wait()
    pltpu.sync_copy(crows_v, cemb_hbm.at[pl.ds(base, EPW)])

    def fire(k, ibuf, rbuf, sem):
        ebase = pl.multiple_of(base + k * CHUNK, CHUNK)
        pltpu.sync_copy(text_hbm.at[pl.ds(ebase, CHUNK)], ibuf)
        for e in range(CHUNK):
            pltpu.async_copy(ttab_hbm.at[ibuf.at[e]],
                             rbuf.at[pl.ds(e * BAG, BAG)], sem)

    def drain(rbuf, sem):
        pltpu.make_async_copy(sum_hbm.at[pl.ds(0, RPC)], rbuf, sem).wait()

    def reduce_chunk(k, rbuf, obuf):
        def elem_body(e, c2):
            r0 = e * BAG
            for d in range(TD // 16):
                acc = rbuf[r0, pl.ds(d * 16, 16)]
                for l in range(1, BAG):
                    acc = acc + rbuf[r0 + l, pl.ds(d * 16, 16)]
                obuf[e, pl.ds(d * 16, 16)] = acc
            return c2

        lax.fori_loop(0, CHUNK, elem_body, 0)
        ebase = pl.multiple_of(base + k * CHUNK, CHUNK)
        pltpu.async_copy(obuf, sum_hbm.at[pl.ds(ebase, CHUNK)], osem)

    fire(0, idx0, rows0, sem0)

    def pipe_body(m, carry):
        k = pl.multiple_of(m * 2, 2)
        fire(k + 1, idx1, rows1, sem1)
        drain(rows0, sem0)
        reduce_chunk(k, rows0, out0)

        @pl.when(m < NCHUNK // 2 - 1)
        def _():
            fire(k + 2, idx0, rows0, sem0)

        drain(rows1, sem1)
        reduce_chunk(k + 1, rows1, out1)
        pltpu.make_async_copy(out0, sum_hbm.at[pl.ds(0, CHUNK)], osem).wait()
        pltpu.make_async_copy(out1, sum_hbm.at[pl.ds(0, CHUNK)], osem).wait()
        return carry

    lax.fori_loop(0, NCHUNK // 2, pipe_body, 0)


_sc_gather = functools.partial(
    pl.kernel,
    out_type=[
        jax.ShapeDtypeStruct((B, TD), jnp.float32),
        jax.ShapeDtypeStruct((B, CD), jnp.float32),
    ],
    mesh=plsc.VectorSubcoreMesh(core_axis_name="c", subcore_axis_name="s"),
    compiler_params=pltpu.CompilerParams(use_tc_tiling_on_sc=False),
    scratch_types=[
        pltpu.VMEM((CHUNK, BAG), jnp.int32),
        pltpu.VMEM((CHUNK, BAG), jnp.int32),
        pltpu.VMEM((RPC, TD), jnp.float32),
        pltpu.VMEM((RPC, TD), jnp.float32),
        pltpu.VMEM((CHUNK, TD), jnp.float32),
        pltpu.VMEM((CHUNK, TD), jnp.float32),
        pltpu.VMEM((EPW // CAT_GB, CAT_GB), jnp.int32),
        pltpu.VMEM((EPW, CD), jnp.float32),
        pltpu.SemaphoreType.DMA,
        pltpu.SemaphoreType.DMA,
        pltpu.SemaphoreType.DMA,
    ],
)(_sc_body)


MLP_BLK = 4096


def _mlp_body(s_ref, c_ref, t_ref, a1_ref, a2_ref, w2_ref, w3_ref,
              b1_ref, b2_ref, b3_ref, o_ref):
    cnt = jnp.sum((t_ref[...] != 0).astype(jnp.float32), axis=1,
                  keepdims=True)
    t = s_ref[...] / jnp.maximum(cnt, 1.0)
    h = jnp.dot(t, a1_ref[...], preferred_element_type=jnp.float32)
    h = h + jnp.dot(c_ref[...], a2_ref[...],
                    preferred_element_type=jnp.float32)
    h = jnp.maximum(h + b1_ref[...], 0.0)
    h = jnp.maximum(
        jnp.dot(h, w2_ref[...], preferred_element_type=jnp.float32)
        + b2_ref[...], 0.0)
    o_ref[...] = (jnp.dot(h, w3_ref[...], preferred_element_type=jnp.float32)
                  + b3_ref[...])


def _mlp(sums, cemb, text, a1, a2, w2t, w3t, b1, b2, b3):
    grid = B // MLP_BLK
    h1 = b1.shape[-1]
    h2 = b2.shape[-1]
    return pl.pallas_call(
        _mlp_body,
        grid=(grid,),
        in_specs=[
            pl.BlockSpec((MLP_BLK, TD), lambda i: (i, 0)),
            pl.BlockSpec((MLP_BLK, CD), lambda i: (i, 0)),
            pl.BlockSpec((MLP_BLK, BAG), lambda i: (i, 0)),
            pl.BlockSpec((TD, h1), lambda i: (0, 0)),
            pl.BlockSpec((CD, h1), lambda i: (0, 0)),
            pl.BlockSpec((h1, h2), lambda i: (0, 0)),
            pl.BlockSpec((h2, TD), lambda i: (0, 0)),
            pl.BlockSpec((1, h1), lambda i: (0, 0)),
            pl.BlockSpec((1, h2), lambda i: (0, 0)),
            pl.BlockSpec((1, TD), lambda i: (0, 0)),
        ],
        out_specs=pl.BlockSpec((MLP_BLK, TD), lambda i: (i, 0)),
        out_shape=jax.ShapeDtypeStruct((B, TD), jnp.float32),
    )(sums, cemb, text, a1, a2, w2t, w3t, b1, b2, b3)


def kernel(text, category, text_table, cat_table, W1, b1, W2, b2, W3, b3):
    text = text.astype(jnp.int32)
    category = category.astype(jnp.int32)
    cat2d = category.reshape(B // CAT_GB, CAT_GB)
    sums, cemb = _sc_gather(text, cat2d, text_table, cat_table)
    a1 = W1.T[:TD, :]
    a2 = W1.T[TD:, :]
    return _mlp(sums, cemb, text, a1, a2, W2.T, W3.T,
                b1.reshape(1, -1), b2.reshape(1, -1), b3.reshape(1, -1))
